# baseline (speedup 1.0000x reference)
import functools

import jax
import jax.numpy as jnp
from jax import lax
from jax.experimental import pallas as pl
from jax.experimental.pallas import tpu as pltpu
from jax.experimental.pallas import tpu_sc as plsc

_T = 32
_WINDOW = 10
_NC = 2
_NS = 16
_L = 16
_CH = 1024
_GROUPS = _CH // _L


def _sigmoid_times(xv):
    sig = 1.0 / (1.0 + jnp.exp(-xv))
    return (sig * float(_WINDOW)).astype(jnp.int32)


def _make_sc_call(B, N):
    n_chunks = N // _CH
    mesh = plsc.VectorSubcoreMesh(core_axis_name="c", subcore_axis_name="s")

    @functools.partial(
        pl.kernel,
        mesh=mesh,
        compiler_params=pltpu.CompilerParams(
            use_tc_tiling_on_sc=False, needs_layout_passes=False
        ),
        out_type=jax.ShapeDtypeStruct((B * _T, N), jnp.float32),
        scratch_types=[
            pltpu.VMEM((_CH,), jnp.float32),
            pltpu.VMEM((_CH,), jnp.float32),
            pltpu.VMEM((_T, _CH), jnp.float32),
            pltpu.VMEM((_T, _CH), jnp.float32),
            pltpu.VMEM((_CH,), jnp.int32),
            pltpu.VMEM((_CH,), jnp.int32),
            pltpu.SemaphoreType.DMA,
            pltpu.SemaphoreType.DMA,
        ],
    )
    def sc_kernel(x_hbm, out_hbm, xa, xb, oa, ob, ta, tb, sa, sb):
        wid = lax.axis_index("s") * _NC + lax.axis_index("c")
        b = wid

        zeros16 = jnp.zeros((_L,), jnp.float32)
        ones16 = jnp.ones((_L,), jnp.float32)
        cols0 = lax.iota(jnp.int32, _L)

        _UNROLL = 4

        def zero_buf(obuf):
            def row(t, c):
                for g in range(_GROUPS):
                    obuf[t, pl.ds(g * _L, _L)] = zeros16
                return c
            lax.fori_loop(0, _T, row, 0)

        def fill_chunk(k, xbuf, obuf, tbuf):
            pltpu.sync_copy(x_hbm.at[b, pl.ds(k * _CH, _CH)], xbuf)

            def body(i, c):
                for u in range(_UNROLL):
                    base = (i * _UNROLL + u) * _L
                    xv = xbuf[pl.ds(base, _L)]
                    times = _sigmoid_times(xv)
                    tbuf[pl.ds(base, _L)] = times
                    plsc.store_scatter(obuf, [times, base + cols0], ones16)
                return c
            lax.fori_loop(0, _GROUPS // _UNROLL, body, 0)

        def clear_chunk(obuf, tbuf):
            def body(i, c):
                for u in range(_UNROLL):
                    base = (i * _UNROLL + u) * _L
                    times = tbuf[pl.ds(base, _L)]
                    plsc.store_scatter(obuf, [times, base + cols0], zeros16)
                return c
            lax.fori_loop(0, _GROUPS // _UNROLL, body, 0)

        def out_slice(k):
            return out_hbm.at[pl.ds(b * _T, _T), pl.ds(k * _CH, _CH)]

        zero_buf(oa)
        zero_buf(ob)

        fill_chunk(0, xa, oa, ta)
        pltpu.async_copy(oa, out_slice(0), sa)
        fill_chunk(1, xb, ob, tb)
        pltpu.async_copy(ob, out_slice(1), sb)

        def pair(i, carry):
            k0 = 2 * i
            pltpu.make_async_copy(oa, out_slice(k0), sa).wait()
            clear_chunk(oa, ta)
            fill_chunk(k0, xa, oa, ta)
            pltpu.async_copy(oa, out_slice(k0), sa)
            k1 = k0 + 1
            pltpu.make_async_copy(ob, out_slice(k1), sb).wait()
            clear_chunk(ob, tb)
            fill_chunk(k1, xb, ob, tb)
            pltpu.async_copy(ob, out_slice(k1), sb)
            return carry

        lax.fori_loop(1, n_chunks // 2, pair, 0)

        pltpu.make_async_copy(oa, out_slice(0), sa).wait()
        pltpu.make_async_copy(ob, out_slice(1), sb).wait()

    return sc_kernel


def kernel(features):
    B, S, F = features.shape
    N = S * F
    x2 = features.reshape(B, N)
    out = _make_sc_call(B, N)(x2)
    return out.reshape(B, _T, S, F)


# speedup vs baseline: 5.7912x; 5.7912x over previous
import jax
import jax.numpy as jnp
from jax.experimental import pallas as pl

_TIMESTEPS = 32
_WINDOW = 10


def _body(x_ref, o_ref):
    x = x_ref[0]
    times = (jax.nn.sigmoid(x) * _WINDOW).astype(jnp.int32)
    t_iota = jax.lax.broadcasted_iota(jnp.int32, (_TIMESTEPS, 1, 1), 0)
    o_ref[0] = (times[None, :, :] == t_iota).astype(jnp.float32)


def kernel(features):
    B, S, F = features.shape
    return pl.pallas_call(
        _body,
        grid=(B,),
        in_specs=[pl.BlockSpec((1, S, F), lambda b: (b, 0, 0))],
        out_specs=pl.BlockSpec((1, _TIMESTEPS, S, F), lambda b: (b, 0, 0, 0)),
        out_shape=jax.ShapeDtypeStruct((B, _TIMESTEPS, S, F), jnp.float32),
    )(features)
